# baseline (speedup 1.0000x reference)
import functools

import jax
import jax.numpy as jnp
from jax import lax
from jax.experimental import pallas as pl
from jax.experimental.pallas import tpu as pltpu
from jax.experimental.pallas import tpu_sc as plsc

MAX_SEQ = 8192
BATCH = 4096
HIST = 200
HIDDEN = 128

CHUNK = 64
NC, NS = 2, 16
NW = NC * NS
N_ROWS = BATCH * HIST
ROWS_PER_W = N_ROWS // NW
NCHUNK = ROWS_PER_W // CHUNK
NBUF = 4
PRIME = 1
OUT_DEPTH = 3

_MESH = plsc.VectorSubcoreMesh(
    core_axis_name="c", subcore_axis_name="s", num_cores=NC, num_subcores=NS)


@functools.partial(
    pl.kernel,
    out_type=jax.ShapeDtypeStruct((N_ROWS, HIDDEN), jnp.float32),
    mesh=_MESH,
    scratch_types=[
        pltpu.VMEM((NCHUNK // 2, 2 * CHUNK), jnp.int32),
        pltpu.VMEM((NBUF, CHUNK, HIDDEN), jnp.float32),
        pltpu.VMEM_SHARED((MAX_SEQ, HIDDEN), jnp.float32),
        pltpu.SemaphoreType.DMA,
        pltpu.SemaphoreType.DMA,
    ],
)
def _embed_sc(x_hbm, table_hbm, out_hbm, idx_v, rows_v, tab_s, gsem, osem):
    sid = lax.axis_index("s")
    wid = sid * NC + lax.axis_index("c")
    base_row = wid * (NCHUNK // 2)
    base_out = wid * ROWS_PER_W

    rows_per_sub = MAX_SEQ // NS
    tcopy = pltpu.async_copy(
        table_hbm.at[pl.ds(sid * rows_per_sub, rows_per_sub)],
        tab_s.at[pl.ds(sid * rows_per_sub, rows_per_sub)], gsem)
    icopy = pltpu.async_copy(x_hbm.at[pl.ds(base_row, NCHUNK // 2)], idx_v, osem)
    tcopy.wait()
    icopy.wait()
    plsc.subcore_barrier()

    def idx_slice(i):
        return idx_v.at[i // 2, pl.ds((i % 2) * CHUNK, CHUNK)]

    def start_gather(i, b):
        pltpu.async_copy(tab_s.at[idx_slice(i)], rows_v.at[b], gsem)

    def wait_gather(i, b):
        pltpu.make_async_copy(tab_s.at[idx_slice(i)], rows_v.at[b], gsem).wait()

    def start_out(i, b):
        pltpu.async_copy(
            rows_v.at[b], out_hbm.at[pl.ds(base_out + i * CHUNK, CHUNK)], osem)

    def wait_out():
        pltpu.make_async_copy(
            rows_v.at[0], out_hbm.at[pl.ds(base_out, CHUNK)], osem).wait()

    for b in range(PRIME):
        start_gather(b, b)

    def full_step(i, b):
        wait_gather(i, b)
        start_out(i, b)
        wait_out()
        start_gather(i + PRIME, (b + PRIME) % NBUF)

    for i in range(NBUF):
        wait_gather(i, i % NBUF)
        start_out(i, i % NBUF)
        if i >= OUT_DEPTH:
            wait_out()
        start_gather(i + PRIME, (i + PRIME) % NBUF)

    def group(g, carry):
        for b in range(NBUF):
            full_step(g * NBUF + b, b)
        return carry

    lax.fori_loop(1, NCHUNK // NBUF - 1, group, 0)

    for i in range(NCHUNK - NBUF, NCHUNK):
        wait_gather(i, i % NBUF)
        start_out(i, i % NBUF)
        wait_out()
        if i + PRIME < NCHUNK:
            start_gather(i + PRIME, (i + PRIME) % NBUF)
    for _ in range(OUT_DEPTH):
        wait_out()


def kernel(x, table):
    x2d = x.reshape(N_ROWS // 128, 128)
    out = _embed_sc(x2d, table)
    return out.reshape(BATCH, HIST, HIDDEN)


# speedup vs baseline: 1.1376x; 1.1376x over previous
import functools

import jax
import jax.numpy as jnp
from jax import lax
from jax.experimental import pallas as pl
from jax.experimental.pallas import tpu as pltpu
from jax.experimental.pallas import tpu_sc as plsc

MAX_SEQ = 8192
BATCH = 4096
HIST = 200
HIDDEN = 128

CHUNK = 32
NC, NS = 2, 16
NW = NC * NS
N_ROWS = BATCH * HIST
ROWS_PER_W = N_ROWS // NW
NCHUNK = ROWS_PER_W // CHUNK
NBUF = 4
PRIME = 2
OUT_DEPTH = 2

_MESH = plsc.VectorSubcoreMesh(
    core_axis_name="c", subcore_axis_name="s", num_cores=NC, num_subcores=NS)


@functools.partial(
    pl.kernel,
    out_type=jax.ShapeDtypeStruct((N_ROWS, HIDDEN), jnp.float32),
    mesh=_MESH,
    scratch_types=[
        pltpu.VMEM((NCHUNK * CHUNK // 128, 128), jnp.int32),
        pltpu.VMEM((NBUF, CHUNK, HIDDEN), jnp.float32),
        pltpu.VMEM_SHARED((MAX_SEQ, HIDDEN), jnp.float32),
        pltpu.SemaphoreType.DMA,
        pltpu.SemaphoreType.DMA,
    ],
)
def _embed_sc(x_hbm, table_hbm, out_hbm, idx_v, rows_v, tab_s, gsem, osem):
    sid = lax.axis_index("s")
    wid = sid * NC + lax.axis_index("c")
    base_row = wid * (NCHUNK * CHUNK // 128)
    base_out = wid * ROWS_PER_W

    rows_per_sub = MAX_SEQ // NS
    tcopy = pltpu.async_copy(
        table_hbm.at[pl.ds(sid * rows_per_sub, rows_per_sub)],
        tab_s.at[pl.ds(sid * rows_per_sub, rows_per_sub)], gsem)
    icopy = pltpu.async_copy(x_hbm.at[pl.ds(base_row, NCHUNK * CHUNK // 128)], idx_v, osem)
    tcopy.wait()
    icopy.wait()
    plsc.subcore_barrier()

    def idx_slice(i):
        per_row = 128 // CHUNK
        return idx_v.at[i // per_row, pl.ds((i % per_row) * CHUNK, CHUNK)]

    def start_gather(i, b):
        pltpu.async_copy(tab_s.at[idx_slice(i)], rows_v.at[b], gsem)

    def wait_gather(i, b):
        pltpu.make_async_copy(tab_s.at[idx_slice(i)], rows_v.at[b], gsem).wait()

    def start_out(i, b):
        pltpu.async_copy(
            rows_v.at[b], out_hbm.at[pl.ds(base_out + i * CHUNK, CHUNK)], osem)

    def wait_out():
        pltpu.make_async_copy(
            rows_v.at[0], out_hbm.at[pl.ds(base_out, CHUNK)], osem).wait()

    for b in range(PRIME):
        start_gather(b, b)

    def full_step(i, b):
        wait_gather(i, b)
        start_out(i, b)
        wait_out()
        start_gather(i + PRIME, (b + PRIME) % NBUF)

    for i in range(NBUF):
        wait_gather(i, i % NBUF)
        start_out(i, i % NBUF)
        if i >= OUT_DEPTH:
            wait_out()
        start_gather(i + PRIME, (i + PRIME) % NBUF)

    def group(g, carry):
        for b in range(NBUF):
            full_step(g * NBUF + b, b)
        return carry

    lax.fori_loop(1, NCHUNK // NBUF - 1, group, 0)

    for i in range(NCHUNK - NBUF, NCHUNK):
        wait_gather(i, i % NBUF)
        start_out(i, i % NBUF)
        wait_out()
        if i + PRIME < NCHUNK:
            start_gather(i + PRIME, (i + PRIME) % NBUF)
    for _ in range(OUT_DEPTH):
        wait_out()


def kernel(x, table):
    x2d = x.reshape(N_ROWS // 128, 128)
    out = _embed_sc(x2d, table)
    return out.reshape(BATCH, HIST, HIDDEN)


# speedup vs baseline: 1.1424x; 1.0042x over previous
import functools

import jax
import jax.numpy as jnp
from jax import lax
from jax.experimental import pallas as pl
from jax.experimental.pallas import tpu as pltpu
from jax.experimental.pallas import tpu_sc as plsc

MAX_SEQ = 8192
BATCH = 4096
HIST = 200
HIDDEN = 128

CHUNK = 32
NC, NS = 2, 16
NW = NC * NS
N_ROWS = BATCH * HIST
ROWS_PER_W = N_ROWS // NW
NCHUNK = ROWS_PER_W // CHUNK
NBUF = 8
PRIME = 4
OUT_DEPTH = 4

_MESH = plsc.VectorSubcoreMesh(
    core_axis_name="c", subcore_axis_name="s", num_cores=NC, num_subcores=NS)


@functools.partial(
    pl.kernel,
    out_type=jax.ShapeDtypeStruct((N_ROWS, HIDDEN), jnp.float32),
    mesh=_MESH,
    scratch_types=[
        pltpu.VMEM((NCHUNK * CHUNK // 128, 128), jnp.int32),
        pltpu.VMEM((NBUF, CHUNK, HIDDEN), jnp.float32),
        pltpu.VMEM_SHARED((MAX_SEQ, HIDDEN), jnp.float32),
        pltpu.SemaphoreType.DMA,
        pltpu.SemaphoreType.DMA,
    ],
)
def _embed_sc(x_hbm, table_hbm, out_hbm, idx_v, rows_v, tab_s, gsem, osem):
    sid = lax.axis_index("s")
    wid = sid * NC + lax.axis_index("c")
    base_row = wid * (NCHUNK * CHUNK // 128)
    base_out = wid * ROWS_PER_W

    rows_per_sub = MAX_SEQ // NS
    tcopy = pltpu.async_copy(
        table_hbm.at[pl.ds(sid * rows_per_sub, rows_per_sub)],
        tab_s.at[pl.ds(sid * rows_per_sub, rows_per_sub)], gsem)
    icopy = pltpu.async_copy(x_hbm.at[pl.ds(base_row, NCHUNK * CHUNK // 128)], idx_v, osem)
    tcopy.wait()
    icopy.wait()
    plsc.subcore_barrier()

    def idx_slice(i):
        per_row = 128 // CHUNK
        return idx_v.at[i // per_row, pl.ds((i % per_row) * CHUNK, CHUNK)]

    def start_gather(i, b):
        pltpu.async_copy(tab_s.at[idx_slice(i)], rows_v.at[b], gsem)

    def wait_gather(i, b):
        pltpu.make_async_copy(tab_s.at[idx_slice(i)], rows_v.at[b], gsem).wait()

    def start_out(i, b):
        pltpu.async_copy(
            rows_v.at[b], out_hbm.at[pl.ds(base_out + i * CHUNK, CHUNK)], osem)

    def wait_out():
        pltpu.make_async_copy(
            rows_v.at[0], out_hbm.at[pl.ds(base_out, CHUNK)], osem).wait()

    for b in range(PRIME):
        start_gather(b, b)

    def full_step(i, b):
        wait_gather(i, b)
        start_out(i, b)
        wait_out()
        start_gather(i + PRIME, (b + PRIME) % NBUF)

    for i in range(NBUF):
        wait_gather(i, i % NBUF)
        start_out(i, i % NBUF)
        if i >= OUT_DEPTH:
            wait_out()
        start_gather(i + PRIME, (i + PRIME) % NBUF)

    def group(g, carry):
        for b in range(NBUF):
            full_step(g * NBUF + b, b)
        return carry

    lax.fori_loop(1, NCHUNK // NBUF - 1, group, 0)

    for i in range(NCHUNK - NBUF, NCHUNK):
        wait_gather(i, i % NBUF)
        start_out(i, i % NBUF)
        wait_out()
        if i + PRIME < NCHUNK:
            start_gather(i + PRIME, (i + PRIME) % NBUF)
    for _ in range(OUT_DEPTH):
        wait_out()


def kernel(x, table):
    x2d = x.reshape(N_ROWS // 128, 128)
    out = _embed_sc(x2d, table)
    return out.reshape(BATCH, HIST, HIDDEN)


# speedup vs baseline: 1.1464x; 1.0036x over previous
import functools

import jax
import jax.numpy as jnp
from jax import lax
from jax.experimental import pallas as pl
from jax.experimental.pallas import tpu as pltpu
from jax.experimental.pallas import tpu_sc as plsc

MAX_SEQ = 8192
BATCH = 4096
HIST = 200
HIDDEN = 128

CHUNK = 32
NC, NS = 2, 16
NW = NC * NS
N_ROWS = BATCH * HIST
ROWS_PER_W = N_ROWS // NW
NCHUNK = ROWS_PER_W // CHUNK
NBUF = 8
PRIME = 5
OUT_DEPTH = 3

_MESH = plsc.VectorSubcoreMesh(
    core_axis_name="c", subcore_axis_name="s", num_cores=NC, num_subcores=NS)


@functools.partial(
    pl.kernel,
    out_type=jax.ShapeDtypeStruct((N_ROWS, HIDDEN), jnp.float32),
    mesh=_MESH,
    scratch_types=[
        pltpu.VMEM((NCHUNK * CHUNK // 128, 128), jnp.int32),
        pltpu.VMEM((NBUF, CHUNK, HIDDEN), jnp.float32),
        pltpu.VMEM_SHARED((MAX_SEQ, HIDDEN), jnp.float32),
        pltpu.SemaphoreType.DMA,
        pltpu.SemaphoreType.DMA,
    ],
)
def _embed_sc(x_hbm, table_hbm, out_hbm, idx_v, rows_v, tab_s, gsem, osem):
    sid = lax.axis_index("s")
    wid = sid * NC + lax.axis_index("c")
    base_row = wid * (NCHUNK * CHUNK // 128)
    base_out = wid * ROWS_PER_W

    rows_per_sub = MAX_SEQ // NS
    tcopy = pltpu.async_copy(
        table_hbm.at[pl.ds(sid * rows_per_sub, rows_per_sub)],
        tab_s.at[pl.ds(sid * rows_per_sub, rows_per_sub)], gsem)
    icopy = pltpu.async_copy(x_hbm.at[pl.ds(base_row, NCHUNK * CHUNK // 128)], idx_v, osem)
    tcopy.wait()
    icopy.wait()
    plsc.subcore_barrier()

    def idx_slice(i):
        per_row = 128 // CHUNK
        return idx_v.at[i // per_row, pl.ds((i % per_row) * CHUNK, CHUNK)]

    def start_gather(i, b):
        pltpu.async_copy(tab_s.at[idx_slice(i)], rows_v.at[b], gsem)

    def wait_gather(i, b):
        pltpu.make_async_copy(tab_s.at[idx_slice(i)], rows_v.at[b], gsem).wait()

    def start_out(i, b):
        pltpu.async_copy(
            rows_v.at[b], out_hbm.at[pl.ds(base_out + i * CHUNK, CHUNK)], osem)

    def wait_out():
        pltpu.make_async_copy(
            rows_v.at[0], out_hbm.at[pl.ds(base_out, CHUNK)], osem).wait()

    for b in range(PRIME):
        start_gather(b, b)

    def full_step(i, b):
        wait_gather(i, b)
        start_out(i, b)
        wait_out()
        start_gather(i + PRIME, (b + PRIME) % NBUF)

    for i in range(NBUF):
        wait_gather(i, i % NBUF)
        start_out(i, i % NBUF)
        if i >= OUT_DEPTH:
            wait_out()
        start_gather(i + PRIME, (i + PRIME) % NBUF)

    def group(g, carry):
        for b in range(NBUF):
            full_step(g * NBUF + b, b)
        return carry

    lax.fori_loop(1, NCHUNK // NBUF - 1, group, 0)

    for i in range(NCHUNK - NBUF, NCHUNK):
        wait_gather(i, i % NBUF)
        start_out(i, i % NBUF)
        wait_out()
        if i + PRIME < NCHUNK:
            start_gather(i + PRIME, (i + PRIME) % NBUF)
    for _ in range(OUT_DEPTH):
        wait_out()


def kernel(x, table):
    x2d = x.reshape(N_ROWS // 128, 128)
    out = _embed_sc(x2d, table)
    return out.reshape(BATCH, HIST, HIDDEN)


# speedup vs baseline: 1.1475x; 1.0009x over previous
import functools

import jax
import jax.numpy as jnp
from jax import lax
from jax.experimental import pallas as pl
from jax.experimental.pallas import tpu as pltpu
from jax.experimental.pallas import tpu_sc as plsc

MAX_SEQ = 8192
BATCH = 4096
HIST = 200
HIDDEN = 128

CHUNK = 32
NC, NS = 2, 16
NW = NC * NS
N_ROWS = BATCH * HIST
ROWS_PER_W = N_ROWS // NW
NCHUNK = ROWS_PER_W // CHUNK
NBUF = 8
PRIME = 4
OUT_DEPTH = 4

_MESH = plsc.VectorSubcoreMesh(
    core_axis_name="c", subcore_axis_name="s", num_cores=NC, num_subcores=NS)


@functools.partial(
    pl.kernel,
    out_type=jax.ShapeDtypeStruct((N_ROWS, HIDDEN), jnp.float32),
    mesh=_MESH,
    scratch_types=[
        pltpu.VMEM((NCHUNK * CHUNK // 128, 128), jnp.int32),
        pltpu.VMEM((NBUF, CHUNK, HIDDEN), jnp.float32),
        pltpu.VMEM_SHARED((MAX_SEQ, HIDDEN), jnp.float32),
        pltpu.SemaphoreType.DMA,
        pltpu.SemaphoreType.DMA,
    ],
)
def _embed_sc(x_hbm, table_hbm, out_hbm, idx_v, rows_v, tab_s, gsem, osem):
    sid = lax.axis_index("s")
    wid = sid * NC + lax.axis_index("c")
    base_row = wid * (NCHUNK * CHUNK // 128)
    base_out = wid * ROWS_PER_W

    rows_per_sub = MAX_SEQ // NS
    tcopy = pltpu.async_copy(
        table_hbm.at[pl.ds(sid * rows_per_sub, rows_per_sub)],
        tab_s.at[pl.ds(sid * rows_per_sub, rows_per_sub)], gsem)
    icopy = pltpu.async_copy(x_hbm.at[pl.ds(base_row, NCHUNK * CHUNK // 128)], idx_v, osem)
    tcopy.wait()
    icopy.wait()
    plsc.subcore_barrier()

    def idx_slice(i):
        per_row = 128 // CHUNK
        return idx_v.at[i // per_row, pl.ds((i % per_row) * CHUNK, CHUNK)]

    def start_gather(i, b):
        pltpu.async_copy(tab_s.at[idx_slice(i)], rows_v.at[b], gsem)

    def wait_gather(i, b):
        pltpu.make_async_copy(tab_s.at[idx_slice(i)], rows_v.at[b], gsem).wait()

    def start_out(i, b):
        pltpu.async_copy(
            rows_v.at[b], out_hbm.at[pl.ds(base_out + i * CHUNK, CHUNK)], osem)

    def wait_out():
        pltpu.make_async_copy(
            rows_v.at[0], out_hbm.at[pl.ds(base_out, CHUNK)], osem).wait()

    for b in range(PRIME):
        start_gather(b, b)

    def full_step(i, b):
        wait_gather(i, b)
        start_out(i, b)
        wait_out()
        start_gather(i + PRIME, (b + PRIME) % NBUF)

    for i in range(NBUF):
        wait_gather(i, i % NBUF)
        start_out(i, i % NBUF)
        if i >= OUT_DEPTH:
            wait_out()
        start_gather(i + PRIME, (i + PRIME) % NBUF)

    def group(g, carry):
        for b in range(NBUF):
            full_step(g * NBUF + b, b)
        return carry

    lax.fori_loop(1, NCHUNK // NBUF - 1, group, 0)

    for i in range(NCHUNK - NBUF, NCHUNK):
        wait_gather(i, i % NBUF)
        start_out(i, i % NBUF)
        wait_out()
        if i + PRIME < NCHUNK:
            start_gather(i + PRIME, (i + PRIME) % NBUF)
    for _ in range(OUT_DEPTH):
        wait_out()


def kernel(x, table):
    x2d = x.reshape(N_ROWS // 128, 128)
    out = _embed_sc(x2d, table)
    return out.reshape(BATCH, HIST, HIDDEN)
